# baseline (speedup 1.0000x reference)
import jax
import jax.numpy as jnp
from jax.experimental import pallas as pl

M_ITEM = 1_000_000
DIM = 64
BLOCK = 32_768


def _dot_block(items_ref, user_ref, out_ref):
    x = items_ref[...]
    u = user_ref[...]
    out_ref[...] = jnp.sum(x * u, axis=-1)


def kernel(items_emb, user_emb):
    n = items_emb.shape[0]
    grid = (n + BLOCK - 1) // BLOCK
    return pl.pallas_call(
        _dot_block,
        grid=(grid,),
        in_specs=[
            pl.BlockSpec((BLOCK, DIM), lambda i: (i, 0)),
            pl.BlockSpec((1, DIM), lambda i: (0, 0)),
        ],
        out_specs=pl.BlockSpec((BLOCK,), lambda i: (i,)),
        out_shape=jax.ShapeDtypeStruct((n,), items_emb.dtype),
    )(items_emb, user_emb)


# speedup vs baseline: 1.5666x; 1.5666x over previous
import jax
import jax.numpy as jnp
from jax.experimental import pallas as pl

M_ITEM = 1_000_000
DIM = 64
BLOCK = 32_768


def _dot_block(items_ref, user_ref, out_ref):
    x = items_ref[...]
    u = user_ref[...]
    x3 = x.reshape(BLOCK // 128, 128, DIM)
    y = jax.lax.dot_general(
        u, x3, (((1,), (2,)), ((), ())), preferred_element_type=jnp.float32
    )
    out_ref[...] = y.reshape(BLOCK)


def kernel(items_emb, user_emb):
    n = items_emb.shape[0]
    grid = (n + BLOCK - 1) // BLOCK
    return pl.pallas_call(
        _dot_block,
        grid=(grid,),
        in_specs=[
            pl.BlockSpec((BLOCK, DIM), lambda i: (i, 0)),
            pl.BlockSpec((1, DIM), lambda i: (0, 0)),
        ],
        out_specs=pl.BlockSpec((BLOCK,), lambda i: (i,)),
        out_shape=jax.ShapeDtypeStruct((n,), items_emb.dtype),
    )(items_emb, user_emb)
